# Initial kernel scaffold; baseline (speedup 1.0000x reference)
import jax
import jax.numpy as jnp
from jax.experimental import pallas as pl


def kernel(x, edge_index, batch, W_gcn, b_gcn, gamma, beta, W_lin, b_lin):
    raise NotImplementedError("write your pallas kernel here")


# speedup vs baseline: 7.8705x; 7.8705x over previous
import functools

import jax
import jax.numpy as jnp
from jax import lax
from jax.experimental import pallas as pl
from jax.experimental.pallas import tpu as pltpu
from jax.experimental.pallas import tpu_sc as plsc

N = 10000
E = 160000
IN_DIM = 256
HID1 = 256
HALF = 128
OUT_DIM = 128
NUM_GRAPHS = 64

NUM_SC = 2
NUM_TILES = 16

E_PER_SC = E // NUM_SC
E_PER_TILE = E_PER_SC // NUM_TILES
CHUNK = 128
NFULL = E_PER_TILE // CHUNK
REM = E_PER_TILE - NFULL * CHUNK
NPAD = 10240
ROWS_PER_TILE = NPAD // NUM_TILES

_mesh = plsc.VectorSubcoreMesh(core_axis_name="c", subcore_axis_name="s")


@functools.partial(
    pl.kernel,
    out_type=jax.ShapeDtypeStruct((NUM_SC, NPAD, HALF), jnp.float32),
    mesh=_mesh,
    scratch_types=[
        pltpu.VMEM((1, CHUNK), jnp.int32),
        pltpu.VMEM((1, REM), jnp.int32),
        pltpu.VMEM((CHUNK, HALF), jnp.float32),
        pltpu.VMEM_SHARED((NPAD, HALF), jnp.float32),
        pltpu.SemaphoreType.DMA,
    ],
)
def _sc_degree(dst_hbm, zeros_hbm, out_hbm, dstv, dstv8, ones_v, acc, sem):
    cc = lax.axis_index("c")
    ss = lax.axis_index("s")

    def fill(i, _):
        def fill_j(j, _):
            ones_v[i, pl.ds(j * 16, 16)] = jnp.full((16,), 1.0, jnp.float32)
            return 0
        lax.fori_loop(0, HALF // 16, fill_j, 0)
        return 0
    lax.fori_loop(0, CHUNK, fill, 0)

    if True:
        row0 = ss * ROWS_PER_TILE
        pltpu.sync_copy(zeros_hbm.at[pl.ds(row0, ROWS_PER_TILE)],
                        acc.at[pl.ds(row0, ROWS_PER_TILE)])
        plsc.subcore_barrier()

        base0 = cc * E_PER_SC + ss * E_PER_TILE

        def body(i, _):
            b = base0 + i * CHUNK
            pltpu.sync_copy(dst_hbm.at[pl.ds(b, CHUNK)], dstv.at[0])
            pltpu.sync_copy(ones_v, acc.at[dstv.at[0]], add=True)
            return 0
        lax.fori_loop(0, NFULL, body, 0)

        bt = base0 + NFULL * CHUNK
        pltpu.sync_copy(dst_hbm.at[pl.ds(bt, REM)], dstv8.at[0])
        pltpu.sync_copy(ones_v.at[pl.ds(0, REM)], acc.at[dstv8.at[0]], add=True)

        plsc.subcore_barrier()
        pltpu.sync_copy(acc.at[pl.ds(row0, ROWS_PER_TILE)],
                        out_hbm.at[cc, pl.ds(row0, ROWS_PER_TILE)])


@functools.partial(
    pl.kernel,
    out_type=jax.ShapeDtypeStruct((NUM_SC, NPAD, HALF), jnp.float32),
    mesh=_mesh,
    scratch_types=[
        pltpu.VMEM((CHUNK,), jnp.int32),
        pltpu.VMEM((1, CHUNK), jnp.int32),
        pltpu.VMEM((REM,), jnp.int32),
        pltpu.VMEM((1, REM), jnp.int32),
        pltpu.VMEM((CHUNK, HALF), jnp.float32),
        pltpu.VMEM((REM, HALF), jnp.float32),
        pltpu.VMEM_SHARED((NPAD, HALF), jnp.float32),
        pltpu.SemaphoreType.DMA,
    ],
)
def _sc_scatter(src_hbm, dst_hbm, table_hbm, zeros_hbm, out_hbm,
                srcv, dstv, srcv8, dstv8, rows, rows8, acc, sem):
    cc = lax.axis_index("c")
    ss = lax.axis_index("s")

    if True:
        row0 = ss * ROWS_PER_TILE
        pltpu.sync_copy(zeros_hbm.at[pl.ds(row0, ROWS_PER_TILE)],
                        acc.at[pl.ds(row0, ROWS_PER_TILE)])
        plsc.subcore_barrier()

        base0 = cc * E_PER_SC + ss * E_PER_TILE

        def body(i, _):
            b = base0 + i * CHUNK
            pltpu.sync_copy(src_hbm.at[pl.ds(b, CHUNK)], srcv)
            pltpu.sync_copy(dst_hbm.at[pl.ds(b, CHUNK)], dstv.at[0])
            pltpu.async_copy(table_hbm.at[srcv], rows, sem).wait()
            pltpu.sync_copy(rows, acc.at[dstv.at[0]], add=True)
            return 0
        lax.fori_loop(0, NFULL, body, 0)

        bt = base0 + NFULL * CHUNK
        pltpu.sync_copy(src_hbm.at[pl.ds(bt, REM)], srcv8)
        pltpu.sync_copy(dst_hbm.at[pl.ds(bt, REM)], dstv8.at[0])
        pltpu.async_copy(table_hbm.at[srcv8], rows8, sem).wait()
        pltpu.sync_copy(rows8, acc.at[dstv8.at[0]], add=True)

        plsc.subcore_barrier()
        pltpu.sync_copy(acc.at[pl.ds(row0, ROWS_PER_TILE)],
                        out_hbm.at[cc, pl.ds(row0, ROWS_PER_TILE)])


_RB = 1000


def _k2_body(x_ref, w_ref, degp_ref, out_ref):
    xw = jnp.dot(x_ref[...], w_ref[...], preferred_element_type=jnp.float32)
    deg = degp_ref[0, :, 0:1] + degp_ref[1, :, 0:1] + 1.0
    dis = lax.rsqrt(deg)
    out_ref[0] = dis * xw[:, :HALF]
    out_ref[1] = dis * xw[:, HALF:]


def _tc_matmul_prescale(x, w, degp):
    return pl.pallas_call(
        _k2_body,
        grid=(N // _RB,),
        in_specs=[
            pl.BlockSpec((_RB, IN_DIM), lambda i: (i, 0)),
            pl.BlockSpec((IN_DIM, HID1), lambda i: (0, 0)),
            pl.BlockSpec((NUM_SC, _RB, HALF), lambda i: (0, i, 0)),
        ],
        out_specs=pl.BlockSpec((NUM_SC, _RB, HALF), lambda i: (0, i, 0)),
        out_shape=jax.ShapeDtypeStruct((NUM_SC, N, HALF), jnp.float32),
    )(x, w, degp)


def _k4_body(plo_ref, phi_ref, xwp_ref, degp_ref, h_ref, stats_ref, acc):
    i = pl.program_id(0)
    deg = degp_ref[0, :, 0:1] + degp_ref[1, :, 0:1] + 1.0
    dis = lax.rsqrt(deg)
    h_lo = dis * (plo_ref[0] + plo_ref[1] + xwp_ref[0])
    h_hi = dis * (phi_ref[0] + phi_ref[1] + xwp_ref[1])
    h = jnp.concatenate([h_lo, h_hi], axis=1)
    h_ref[...] = h

    @pl.when(i == 0)
    def _():
        acc[...] = jnp.zeros_like(acc)

    s = jnp.sum(h, axis=0, keepdims=True)
    s2 = jnp.sum(h * h, axis=0, keepdims=True)
    acc[0:1, :] += s
    acc[1:2, :] += s2
    stats_ref[...] = acc[...]


def _tc_merge(plo, phi, xwp, degp):
    return pl.pallas_call(
        _k4_body,
        grid=(N // _RB,),
        in_specs=[
            pl.BlockSpec((NUM_SC, _RB, HALF), lambda i: (0, i, 0)),
            pl.BlockSpec((NUM_SC, _RB, HALF), lambda i: (0, i, 0)),
            pl.BlockSpec((NUM_SC, _RB, HALF), lambda i: (0, i, 0)),
            pl.BlockSpec((NUM_SC, _RB, HALF), lambda i: (0, i, 0)),
        ],
        out_specs=[
            pl.BlockSpec((_RB, HID1), lambda i: (i, 0)),
            pl.BlockSpec((8, HID1), lambda i: (0, 0)),
        ],
        out_shape=[
            jax.ShapeDtypeStruct((N, HID1), jnp.float32),
            jax.ShapeDtypeStruct((8, HID1), jnp.float32),
        ],
        scratch_shapes=[pltpu.VMEM((8, HID1), jnp.float32)],
    )(plo, phi, xwp, degp)


def _k5_body(h_ref, stats_ref, gb_ref, batch_s_ref, batch_v_ref,
             wlin_ref, blin_ref, out_ref, pooled):
    i = pl.program_id(0)
    nb = pl.num_programs(0)

    mean = stats_ref[0:1, :] / float(N)
    var = stats_ref[1:2, :] / float(N) - mean * mean
    gamma = gb_ref[0:1, :]
    beta = gb_ref[1:2, :]
    scale = gamma * lax.rsqrt(var + 1e-5)
    shift = beta - mean * scale
    hn = jnp.maximum(h_ref[...] * scale + shift, 0.0)

    @pl.when(i == 0)
    def _():
        pooled[...] = jnp.zeros_like(pooled)

    g_lo = batch_s_ref[0, 0]
    g_hi = batch_s_ref[_RB - 1, 0]
    bcol = batch_v_ref[...]

    def seg(g, _):
        m = bcol == g
        masked = jnp.where(m, hn, -1e30)
        bm = jnp.max(masked, axis=0, keepdims=True)
        cur = pooled[pl.ds(g, 1), :]
        pooled[pl.ds(g, 1), :] = jnp.maximum(cur, bm)
        return 0
    lax.fori_loop(g_lo, g_hi + 1, seg, 0)

    @pl.when(i == nb - 1)
    def _():
        out_ref[...] = (
            jnp.dot(pooled[...], wlin_ref[...],
                    preferred_element_type=jnp.float32) + blin_ref[...])


def _tc_final(h, stats, gb, batch, wlin, blin):
    batch2 = batch.reshape(N, 1)
    return pl.pallas_call(
        _k5_body,
        grid=(N // _RB,),
        in_specs=[
            pl.BlockSpec((_RB, HID1), lambda i: (i, 0)),
            pl.BlockSpec((8, HID1), lambda i: (0, 0)),
            pl.BlockSpec((2, HID1), lambda i: (0, 0)),
            pl.BlockSpec((_RB, 1), lambda i: (i, 0),
                         memory_space=pltpu.SMEM),
            pl.BlockSpec((_RB, 1), lambda i: (i, 0)),
            pl.BlockSpec((HID1, OUT_DIM), lambda i: (0, 0)),
            pl.BlockSpec((1, OUT_DIM), lambda i: (0, 0)),
        ],
        out_specs=pl.BlockSpec((NUM_GRAPHS, OUT_DIM), lambda i: (0, 0)),
        out_shape=jax.ShapeDtypeStruct((NUM_GRAPHS, OUT_DIM), jnp.float32),
        scratch_shapes=[pltpu.VMEM((NUM_GRAPHS, HID1), jnp.float32)],
    )(h, stats, gb, batch2, batch2, wlin, blin)


def kernel(x, edge_index, batch, W_gcn, b_gcn, gamma, beta, W_lin, b_lin):
    src = edge_index[0]
    dst = edge_index[1]
    zeros128 = jnp.zeros((NPAD, HALF), jnp.float32)

    degp = _sc_degree(dst, zeros128)
    xwp = _tc_matmul_prescale(x, W_gcn, degp)
    plo = _sc_scatter(src, dst, xwp[0], zeros128)
    phi = _sc_scatter(src, dst, xwp[1], zeros128)
    h, stats = _tc_merge(plo, phi, xwp, degp)
    gb = jnp.stack([gamma, beta])
    out = _tc_final(h, stats, gb, batch, W_lin, b_lin.reshape(1, OUT_DIM))
    return out
